# baseline (speedup 1.0000x reference)
import jax
import jax.numpy as jnp
from jax.experimental import pallas as pl
from jax.experimental.pallas import tpu as pltpu

N = 10000
F = 128
BM = 400
NI = N // BM


def _norm_body(x_ref, xn_ref):
    x = x_ref[...]
    xn_ref[...] = x / (jnp.sqrt(jnp.sum(x * x, axis=-1, keepdims=True)) + 1e-8)


def _pass1_body(adj_ref, xni_ref, xnf_ref, xf_ref, w_ref, b_ref,
                mask_ref, h1hi_ref, h1lo_ref, rdeg_ref):
    sim = jax.lax.dot_general(
        xni_ref[...], xnf_ref[...], (((1,), (1,)), ((), ())),
        preferred_element_type=jnp.float32)
    m = (adj_ref[...] > 0.5) & (sim > 0.5)
    mf = m.astype(jnp.float32)
    mask_ref[...] = m.astype(jnp.int8)
    rdeg = 1.0 / jnp.maximum(jnp.sum(mf, axis=1, keepdims=True), 1.0)
    u = jnp.dot(mf, xf_ref[...], preferred_element_type=jnp.float32) * rdeg
    h1 = jnp.maximum(
        jnp.dot(u, w_ref[...], preferred_element_type=jnp.float32)
        + b_ref[...], 0.0)
    hi = h1.astype(jnp.bfloat16)
    h1hi_ref[...] = hi
    h1lo_ref[...] = (h1 - hi.astype(jnp.float32)).astype(jnp.bfloat16)
    rdeg_ref[...] = rdeg


def _pass2_body(mask_ref, h1hi_ref, h1lo_ref, rdeg_ref, w_ref, b_ref, out_ref):
    mf = mask_ref[...].astype(jnp.bfloat16)
    u = (jnp.dot(mf, h1hi_ref[...], preferred_element_type=jnp.float32)
         + jnp.dot(mf, h1lo_ref[...], preferred_element_type=jnp.float32))
    u = u * rdeg_ref[...]
    h2 = jnp.maximum(
        jnp.dot(u, w_ref[...], preferred_element_type=jnp.float32)
        + b_ref[...], 0.0)
    z = h2 - jnp.max(h2, axis=-1, keepdims=True)
    e = jnp.exp(z)
    out_ref[...] = e / jnp.sum(e, axis=-1, keepdims=True)


def kernel(adj_matrix, transaction_record, labels, W, b):
    x = transaction_record
    b2 = b.reshape(1, F)

    xn = pl.pallas_call(
        _norm_body,
        out_shape=jax.ShapeDtypeStruct((N, F), jnp.float32),
    )(x)

    mask, h1hi, h1lo, rdeg = pl.pallas_call(
        _pass1_body,
        grid=(NI,),
        in_specs=[
            pl.BlockSpec((BM, N), lambda i: (i, 0)),
            pl.BlockSpec((BM, F), lambda i: (i, 0)),
            pl.BlockSpec((N, F), lambda i: (0, 0)),
            pl.BlockSpec((N, F), lambda i: (0, 0)),
            pl.BlockSpec((F, F), lambda i: (0, 0)),
            pl.BlockSpec((1, F), lambda i: (0, 0)),
        ],
        out_specs=[
            pl.BlockSpec((BM, N), lambda i: (i, 0)),
            pl.BlockSpec((BM, F), lambda i: (i, 0)),
            pl.BlockSpec((BM, F), lambda i: (i, 0)),
            pl.BlockSpec((BM, 1), lambda i: (i, 0)),
        ],
        out_shape=[
            jax.ShapeDtypeStruct((N, N), jnp.int8),
            jax.ShapeDtypeStruct((N, F), jnp.bfloat16),
            jax.ShapeDtypeStruct((N, F), jnp.bfloat16),
            jax.ShapeDtypeStruct((N, 1), jnp.float32),
        ],
        compiler_params=pltpu.CompilerParams(
            dimension_semantics=("arbitrary",)),
    )(adj_matrix, xn, xn, x, W, b2)

    out = pl.pallas_call(
        _pass2_body,
        grid=(NI,),
        in_specs=[
            pl.BlockSpec((BM, N), lambda i: (i, 0)),
            pl.BlockSpec((N, F), lambda i: (0, 0)),
            pl.BlockSpec((N, F), lambda i: (0, 0)),
            pl.BlockSpec((BM, 1), lambda i: (i, 0)),
            pl.BlockSpec((F, F), lambda i: (0, 0)),
            pl.BlockSpec((1, F), lambda i: (0, 0)),
        ],
        out_specs=pl.BlockSpec((BM, F), lambda i: (i, 0)),
        out_shape=jax.ShapeDtypeStruct((N, F), jnp.float32),
        compiler_params=pltpu.CompilerParams(
            dimension_semantics=("arbitrary",)),
    )(mask, h1hi, h1lo, rdeg, W, b2)

    return out


# speedup vs baseline: 1.1955x; 1.1955x over previous
import jax
import jax.numpy as jnp
from jax.experimental import pallas as pl
from jax.experimental.pallas import tpu as pltpu

N = 10000
F = 128
BM = 400
NI = N // BM


def _norm_body(x_ref, xn_ref):
    x = x_ref[...]
    xn_ref[...] = x / (jnp.sqrt(jnp.sum(x * x, axis=-1, keepdims=True)) + 1e-8)


def _pass1_body(adj_ref, xni_ref, xnf_ref, xf_ref, w_ref, b_ref,
                mask_ref, h1_ref, rdeg_ref):
    sim = jax.lax.dot_general(
        xni_ref[...], xnf_ref[...], (((1,), (1,)), ((), ())),
        preferred_element_type=jnp.float32)
    m = (adj_ref[...] > 0.5) & (sim > 0.5)
    mf = m.astype(jnp.float32)
    mask_ref[...] = m.astype(jnp.int8)
    rdeg = 1.0 / jnp.maximum(jnp.sum(mf, axis=1, keepdims=True), 1.0)
    u = jnp.dot(mf, xf_ref[...], preferred_element_type=jnp.float32) * rdeg
    h1 = jnp.maximum(
        jnp.dot(u, w_ref[...], preferred_element_type=jnp.float32)
        + b_ref[...], 0.0)
    h1_ref[...] = h1
    rdeg_ref[...] = rdeg


def _pass2_body(mask_ref, h1_ref, rdeg_ref, w_ref, b_ref, out_ref):
    mf = mask_ref[...].astype(jnp.float32)
    u = jnp.dot(mf, h1_ref[...], preferred_element_type=jnp.float32)
    u = u * rdeg_ref[...]
    h2 = jnp.maximum(
        jnp.dot(u, w_ref[...], preferred_element_type=jnp.float32)
        + b_ref[...], 0.0)
    z = h2 - jnp.max(h2, axis=-1, keepdims=True)
    e = jnp.exp(z)
    out_ref[...] = e / jnp.sum(e, axis=-1, keepdims=True)


def kernel(adj_matrix, transaction_record, labels, W, b):
    x = transaction_record
    b2 = b.reshape(1, F)

    xn = pl.pallas_call(
        _norm_body,
        out_shape=jax.ShapeDtypeStruct((N, F), jnp.float32),
    )(x)

    mask, h1, rdeg = pl.pallas_call(
        _pass1_body,
        grid=(NI,),
        in_specs=[
            pl.BlockSpec((BM, N), lambda i: (i, 0)),
            pl.BlockSpec((BM, F), lambda i: (i, 0)),
            pl.BlockSpec((N, F), lambda i: (0, 0)),
            pl.BlockSpec((N, F), lambda i: (0, 0)),
            pl.BlockSpec((F, F), lambda i: (0, 0)),
            pl.BlockSpec((1, F), lambda i: (0, 0)),
        ],
        out_specs=[
            pl.BlockSpec((BM, N), lambda i: (i, 0)),
            pl.BlockSpec((BM, F), lambda i: (i, 0)),
            pl.BlockSpec((BM, 1), lambda i: (i, 0)),
        ],
        out_shape=[
            jax.ShapeDtypeStruct((N, N), jnp.int8),
            jax.ShapeDtypeStruct((N, F), jnp.float32),
            jax.ShapeDtypeStruct((N, 1), jnp.float32),
        ],
        compiler_params=pltpu.CompilerParams(
            dimension_semantics=("arbitrary",)),
    )(adj_matrix, xn, xn, x, W, b2)

    out = pl.pallas_call(
        _pass2_body,
        grid=(NI,),
        in_specs=[
            pl.BlockSpec((BM, N), lambda i: (i, 0)),
            pl.BlockSpec((N, F), lambda i: (0, 0)),
            pl.BlockSpec((BM, 1), lambda i: (i, 0)),
            pl.BlockSpec((F, F), lambda i: (0, 0)),
            pl.BlockSpec((1, F), lambda i: (0, 0)),
        ],
        out_specs=pl.BlockSpec((BM, F), lambda i: (i, 0)),
        out_shape=jax.ShapeDtypeStruct((N, F), jnp.float32),
        compiler_params=pltpu.CompilerParams(
            dimension_semantics=("arbitrary",)),
    )(mask, h1, rdeg, W, b2)

    return out


# speedup vs baseline: 1.2508x; 1.0462x over previous
import jax
import jax.numpy as jnp
from jax.experimental import pallas as pl
from jax.experimental.pallas import tpu as pltpu

N = 10000
F = 128
BM = 400
NI = N // BM


def _norm_body(x_ref, xn_ref):
    x = x_ref[...]
    xn_ref[...] = x / (jnp.sqrt(jnp.sum(x * x, axis=-1, keepdims=True)) + 1e-8)


def _pass1_body(adj_ref, xni_ref, xnf_ref, xf_ref, w_ref, b_ref,
                mask_ref, h1_ref, rdeg_ref):
    sim = jax.lax.dot_general(
        xni_ref[...], xnf_ref[...], (((1,), (1,)), ((), ())),
        preferred_element_type=jnp.float32)
    m = jnp.minimum(adj_ref[...], sim) > 0.5
    mf = jnp.where(m, 1.0, 0.0)
    mask_ref[...] = mf.astype(jnp.int8)
    rdeg = 1.0 / jnp.maximum(jnp.sum(mf, axis=1, keepdims=True), 1.0)
    u = jnp.dot(mf, xf_ref[...], preferred_element_type=jnp.float32)
    h1 = jnp.maximum(
        jnp.dot(u, w_ref[...], preferred_element_type=jnp.float32) * rdeg
        + b_ref[...], 0.0)
    h1_ref[...] = h1
    rdeg_ref[...] = rdeg


def _pass2_body(mask_ref, h1_ref, rdeg_ref, w_ref, b_ref, out_ref):
    mf = mask_ref[...].astype(jnp.float32)
    u = jnp.dot(mf, h1_ref[...], preferred_element_type=jnp.float32)
    u = u * rdeg_ref[...]
    h2 = jnp.maximum(
        jnp.dot(u, w_ref[...], preferred_element_type=jnp.float32)
        + b_ref[...], 0.0)
    z = h2 - jnp.max(h2, axis=-1, keepdims=True)
    e = jnp.exp(z)
    out_ref[...] = e / jnp.sum(e, axis=-1, keepdims=True)


def kernel(adj_matrix, transaction_record, labels, W, b):
    x = transaction_record
    b2 = b.reshape(1, F)

    xn = pl.pallas_call(
        _norm_body,
        out_shape=jax.ShapeDtypeStruct((N, F), jnp.float32),
    )(x)

    mask, h1, rdeg = pl.pallas_call(
        _pass1_body,
        grid=(NI,),
        in_specs=[
            pl.BlockSpec((BM, N), lambda i: (i, 0)),
            pl.BlockSpec((BM, F), lambda i: (i, 0)),
            pl.BlockSpec((N, F), lambda i: (0, 0)),
            pl.BlockSpec((N, F), lambda i: (0, 0)),
            pl.BlockSpec((F, F), lambda i: (0, 0)),
            pl.BlockSpec((1, F), lambda i: (0, 0)),
        ],
        out_specs=[
            pl.BlockSpec((BM, N), lambda i: (i, 0)),
            pl.BlockSpec((BM, F), lambda i: (i, 0)),
            pl.BlockSpec((BM, 1), lambda i: (i, 0)),
        ],
        out_shape=[
            jax.ShapeDtypeStruct((N, N), jnp.int8),
            jax.ShapeDtypeStruct((N, F), jnp.float32),
            jax.ShapeDtypeStruct((N, 1), jnp.float32),
        ],
        compiler_params=pltpu.CompilerParams(
            dimension_semantics=("parallel",)),
    )(adj_matrix, xn, xn, x, W, b2)

    out = pl.pallas_call(
        _pass2_body,
        grid=(NI,),
        in_specs=[
            pl.BlockSpec((BM, N), lambda i: (i, 0)),
            pl.BlockSpec((N, F), lambda i: (0, 0)),
            pl.BlockSpec((BM, 1), lambda i: (i, 0)),
            pl.BlockSpec((F, F), lambda i: (0, 0)),
            pl.BlockSpec((1, F), lambda i: (0, 0)),
        ],
        out_specs=pl.BlockSpec((BM, F), lambda i: (i, 0)),
        out_shape=jax.ShapeDtypeStruct((N, F), jnp.float32),
        compiler_params=pltpu.CompilerParams(
            dimension_semantics=("parallel",)),
    )(mask, h1, rdeg, W, b2)

    return out


# speedup vs baseline: 1.2579x; 1.0057x over previous
import jax
import jax.numpy as jnp
from jax.experimental import pallas as pl
from jax.experimental.pallas import tpu as pltpu

N = 10000
F = 128
BM = 400
NI = N // BM


def _norm_body(x_ref, xn_ref):
    x = x_ref[...]
    xn_ref[...] = x / (jnp.sqrt(jnp.sum(x * x, axis=-1, keepdims=True)) + 1e-8)


def _pass1_body(adj_ref, xni_ref, xnf_ref, xf_ref, w_ref, b_ref,
                mask_ref, h1_ref, rdeg_ref):
    sim = jax.lax.dot_general(
        xni_ref[...], xnf_ref[...], (((1,), (1,)), ((), ())),
        preferred_element_type=jnp.float32)
    m = jnp.minimum(adj_ref[...], sim) > 0.5
    mf = jnp.where(m, 1.0, 0.0)
    mask_ref[...] = mf.astype(jnp.int8)
    rdeg = 1.0 / jnp.maximum(jnp.sum(mf, axis=1, keepdims=True), 1.0)
    u = jnp.dot(mf, xf_ref[...], preferred_element_type=jnp.float32)
    h1 = jnp.maximum(
        jnp.dot(u, w_ref[...], preferred_element_type=jnp.float32) * rdeg
        + b_ref[...], 0.0)
    h1_ref[...] = h1.astype(jnp.bfloat16)
    rdeg_ref[...] = rdeg


def _pass2_body(mask_ref, h1_ref, rdeg_ref, w_ref, b_ref, out_ref):
    mf = mask_ref[...].astype(jnp.bfloat16)
    u = jnp.dot(mf, h1_ref[...], preferred_element_type=jnp.float32)
    h2 = jnp.maximum(
        jnp.dot(u, w_ref[...], preferred_element_type=jnp.float32)
        * rdeg_ref[...] + b_ref[...], 0.0)
    z = h2 - jnp.max(h2, axis=-1, keepdims=True)
    e = jnp.exp(z)
    out_ref[...] = e / jnp.sum(e, axis=-1, keepdims=True)


def kernel(adj_matrix, transaction_record, labels, W, b):
    x = transaction_record
    b2 = b.reshape(1, F)

    xn = pl.pallas_call(
        _norm_body,
        out_shape=jax.ShapeDtypeStruct((N, F), jnp.float32),
    )(x)

    mask, h1, rdeg = pl.pallas_call(
        _pass1_body,
        grid=(NI,),
        in_specs=[
            pl.BlockSpec((BM, N), lambda i: (i, 0)),
            pl.BlockSpec((BM, F), lambda i: (i, 0)),
            pl.BlockSpec((N, F), lambda i: (0, 0)),
            pl.BlockSpec((N, F), lambda i: (0, 0)),
            pl.BlockSpec((F, F), lambda i: (0, 0)),
            pl.BlockSpec((1, F), lambda i: (0, 0)),
        ],
        out_specs=[
            pl.BlockSpec((BM, N), lambda i: (i, 0)),
            pl.BlockSpec((BM, F), lambda i: (i, 0)),
            pl.BlockSpec((BM, 1), lambda i: (i, 0)),
        ],
        out_shape=[
            jax.ShapeDtypeStruct((N, N), jnp.int8),
            jax.ShapeDtypeStruct((N, F), jnp.bfloat16),
            jax.ShapeDtypeStruct((N, 1), jnp.float32),
        ],
        compiler_params=pltpu.CompilerParams(
            dimension_semantics=("parallel",)),
    )(adj_matrix, xn, xn, x, W, b2)

    out = pl.pallas_call(
        _pass2_body,
        grid=(NI,),
        in_specs=[
            pl.BlockSpec((BM, N), lambda i: (i, 0)),
            pl.BlockSpec((N, F), lambda i: (0, 0)),
            pl.BlockSpec((BM, 1), lambda i: (i, 0)),
            pl.BlockSpec((F, F), lambda i: (0, 0)),
            pl.BlockSpec((1, F), lambda i: (0, 0)),
        ],
        out_specs=pl.BlockSpec((BM, F), lambda i: (i, 0)),
        out_shape=jax.ShapeDtypeStruct((N, F), jnp.float32),
        compiler_params=pltpu.CompilerParams(
            dimension_semantics=("parallel",)),
    )(mask, h1, rdeg, W, b2)

    return out


# speedup vs baseline: 1.2653x; 1.0058x over previous
import jax
import jax.numpy as jnp
from jax.experimental import pallas as pl
from jax.experimental.pallas import tpu as pltpu

N = 10000
F = 128
BM = 400
NI = N // BM


def _norm_body(x_ref, xn_ref):
    x = x_ref[...]
    xn_ref[...] = x / (jnp.sqrt(jnp.sum(x * x, axis=-1, keepdims=True)) + 1e-8)


def _pass1_body(adj_ref, xni_ref, xnf_ref, xf_ref, w_ref, b_ref,
                mask_ref, h1_ref, rdeg_ref):
    sim = jax.lax.dot_general(
        xni_ref[...], xnf_ref[...], (((1,), (1,)), ((), ())),
        preferred_element_type=jnp.float32)
    m = jnp.minimum(adj_ref[...], sim) > 0.5
    mf = jnp.where(m, 1.0, 0.0)
    mask_ref[...] = mf.astype(jnp.float8_e4m3fn)
    rdeg = 1.0 / jnp.maximum(jnp.sum(mf, axis=1, keepdims=True), 1.0)
    u = jnp.dot(mf, xf_ref[...], preferred_element_type=jnp.float32)
    h1 = jnp.maximum(
        jnp.dot(u, w_ref[...], preferred_element_type=jnp.float32) * rdeg
        + b_ref[...], 0.0)
    h1_ref[...] = h1.astype(jnp.bfloat16)
    rdeg_ref[...] = rdeg


def _pass2_body(mask_ref, h1_ref, rdeg_ref, w_ref, b_ref, out_ref):
    u = jnp.dot(mask_ref[...].astype(jnp.bfloat16), h1_ref[...],
                preferred_element_type=jnp.float32)
    h2 = jnp.maximum(
        jnp.dot(u, w_ref[...], preferred_element_type=jnp.float32)
        * rdeg_ref[...] + b_ref[...], 0.0)
    z = h2 - jnp.max(h2, axis=-1, keepdims=True)
    e = jnp.exp(z)
    out_ref[...] = e / jnp.sum(e, axis=-1, keepdims=True)


def kernel(adj_matrix, transaction_record, labels, W, b):
    x = transaction_record
    b2 = b.reshape(1, F)

    xn = pl.pallas_call(
        _norm_body,
        out_shape=jax.ShapeDtypeStruct((N, F), jnp.float32),
    )(x)

    mask, h1, rdeg = pl.pallas_call(
        _pass1_body,
        grid=(NI,),
        in_specs=[
            pl.BlockSpec((BM, N), lambda i: (i, 0)),
            pl.BlockSpec((BM, F), lambda i: (i, 0)),
            pl.BlockSpec((N, F), lambda i: (0, 0)),
            pl.BlockSpec((N, F), lambda i: (0, 0)),
            pl.BlockSpec((F, F), lambda i: (0, 0)),
            pl.BlockSpec((1, F), lambda i: (0, 0)),
        ],
        out_specs=[
            pl.BlockSpec((BM, N), lambda i: (i, 0)),
            pl.BlockSpec((BM, F), lambda i: (i, 0)),
            pl.BlockSpec((BM, 1), lambda i: (i, 0)),
        ],
        out_shape=[
            jax.ShapeDtypeStruct((N, N), jnp.float8_e4m3fn),
            jax.ShapeDtypeStruct((N, F), jnp.bfloat16),
            jax.ShapeDtypeStruct((N, 1), jnp.float32),
        ],
        compiler_params=pltpu.CompilerParams(
            dimension_semantics=("parallel",)),
    )(adj_matrix, xn, xn, x, W, b2)

    out = pl.pallas_call(
        _pass2_body,
        grid=(NI,),
        in_specs=[
            pl.BlockSpec((BM, N), lambda i: (i, 0)),
            pl.BlockSpec((N, F), lambda i: (0, 0)),
            pl.BlockSpec((BM, 1), lambda i: (i, 0)),
            pl.BlockSpec((F, F), lambda i: (0, 0)),
            pl.BlockSpec((1, F), lambda i: (0, 0)),
        ],
        out_specs=pl.BlockSpec((BM, F), lambda i: (i, 0)),
        out_shape=jax.ShapeDtypeStruct((N, F), jnp.float32),
        compiler_params=pltpu.CompilerParams(
            dimension_semantics=("parallel",)),
    )(mask, h1, rdeg, W, b2)

    return out


# speedup vs baseline: 1.3720x; 1.0844x over previous
import jax
import jax.numpy as jnp
from jax.experimental import pallas as pl
from jax.experimental.pallas import tpu as pltpu

N = 10000
F = 128
BM = 400
NI = N // BM


def _norm_body(x_ref, xn_ref):
    x = x_ref[...]
    xn_ref[...] = x / (jnp.sqrt(jnp.sum(x * x, axis=-1, keepdims=True)) + 1e-8)


def _pass1_body(adj_ref, xni_ref, xnf_ref, xf_ref, w_ref, b_ref,
                mask_ref, h1_ref, rdeg_ref):
    sim = jax.lax.dot_general(
        xni_ref[...], xnf_ref[...], (((1,), (1,)), ((), ())),
        preferred_element_type=jnp.float32)
    m = jnp.minimum(adj_ref[...], sim) > 0.5
    mf = jnp.where(m, 1.0, 0.0)
    mask_ref[...] = mf.astype(jnp.int4)
    rdeg = 1.0 / jnp.maximum(jnp.sum(mf, axis=1, keepdims=True), 1.0)
    u = jnp.dot(mf, xf_ref[...], preferred_element_type=jnp.float32)
    h1 = jnp.maximum(
        jnp.dot(u, w_ref[...], preferred_element_type=jnp.float32) * rdeg
        + b_ref[...], 0.0)
    h1_ref[...] = h1.astype(jnp.bfloat16)
    rdeg_ref[...] = rdeg


def _pass2_body(mask_ref, h1_ref, rdeg_ref, w_ref, b_ref, out_ref):
    u = jnp.dot(mask_ref[...].astype(jnp.bfloat16), h1_ref[...],
                preferred_element_type=jnp.float32)
    h2 = jnp.maximum(
        jnp.dot(u, w_ref[...], preferred_element_type=jnp.float32)
        * rdeg_ref[...] + b_ref[...], 0.0)
    z = h2 - jnp.max(h2, axis=-1, keepdims=True)
    e = jnp.exp(z)
    out_ref[...] = e / jnp.sum(e, axis=-1, keepdims=True)


def kernel(adj_matrix, transaction_record, labels, W, b):
    x = transaction_record
    b2 = b.reshape(1, F)

    xn = pl.pallas_call(
        _norm_body,
        out_shape=jax.ShapeDtypeStruct((N, F), jnp.float32),
    )(x)

    mask, h1, rdeg = pl.pallas_call(
        _pass1_body,
        grid=(NI,),
        in_specs=[
            pl.BlockSpec((BM, N), lambda i: (i, 0)),
            pl.BlockSpec((BM, F), lambda i: (i, 0)),
            pl.BlockSpec((N, F), lambda i: (0, 0)),
            pl.BlockSpec((N, F), lambda i: (0, 0)),
            pl.BlockSpec((F, F), lambda i: (0, 0)),
            pl.BlockSpec((1, F), lambda i: (0, 0)),
        ],
        out_specs=[
            pl.BlockSpec((BM, N), lambda i: (i, 0)),
            pl.BlockSpec((BM, F), lambda i: (i, 0)),
            pl.BlockSpec((BM, 1), lambda i: (i, 0)),
        ],
        out_shape=[
            jax.ShapeDtypeStruct((N, N), jnp.int4),
            jax.ShapeDtypeStruct((N, F), jnp.bfloat16),
            jax.ShapeDtypeStruct((N, 1), jnp.float32),
        ],
        compiler_params=pltpu.CompilerParams(
            dimension_semantics=("parallel",)),
    )(adj_matrix, xn, xn, x, W, b2)

    out = pl.pallas_call(
        _pass2_body,
        grid=(NI,),
        in_specs=[
            pl.BlockSpec((BM, N), lambda i: (i, 0)),
            pl.BlockSpec((N, F), lambda i: (0, 0)),
            pl.BlockSpec((BM, 1), lambda i: (i, 0)),
            pl.BlockSpec((F, F), lambda i: (0, 0)),
            pl.BlockSpec((1, F), lambda i: (0, 0)),
        ],
        out_specs=pl.BlockSpec((BM, F), lambda i: (i, 0)),
        out_shape=jax.ShapeDtypeStruct((N, F), jnp.float32),
        compiler_params=pltpu.CompilerParams(
            dimension_semantics=("parallel",)),
    )(mask, h1, rdeg, W, b2)

    return out
